# baseline (speedup 1.0000x reference)
import jax
import jax.numpy as jnp
from jax import lax
from jax.experimental import pallas as pl
from jax.experimental.pallas import tpu as pltpu
from jax.experimental.pallas import tpu_sc as plsc

N = 10000
NP = 10240
ET = 172032
HF = 256
W = 128
NC, NS = 2, 16

_MESH = plsc.VectorSubcoreMesh(core_axis_name="c", subcore_axis_name="s")
_SC_PARAMS = pltpu.CompilerParams(use_tc_tiling_on_sc=False)
_f32 = jnp.float32


def _leaky(x):
    return jnp.where(x >= 0, x, 0.2 * x)


def _elu(x):
    return jnp.where(x > 0, x, jnp.exp(x) - 1.0)


def _sc_gather_fused(tables, indices, widths, dtypes):
    GW = 128
    n = len(tables)
    out_type = [jax.ShapeDtypeStruct((ET, w), dt)
                for w, dt in zip(widths, dtypes)]

    def body(*refs):
        t_hbms = refs[:n]
        i_hbms = refs[n:2 * n]
        o_hbms = refs[2 * n:3 * n]
        sem = refs[3 * n]

        def inner(*bufs):
            idx_vs = bufs[:n]
            out_vs = bufs[n:]
            copies = [pltpu.async_copy(t_hbms[k].at[idx_vs[k].at[0]],
                                       out_vs[k], sem) for k in range(n)]
            for cp in copies:
                cp.wait()

        pltpu.emit_pipeline(
            inner,
            grid=(ET // GW,),
            in_specs=[pl.BlockSpec((1, GW), lambda i: (0, i))] * n,
            out_specs=[pl.BlockSpec((GW, w), lambda i: (i, 0))
                       for w in widths],
            core_axis_name=("c", "s"),
            dimension_semantics=(pltpu.PARALLEL,),
        )(*i_hbms, *o_hbms)

    return pl.kernel(body, out_type=out_type, mesh=_MESH,
                     compiler_params=_SC_PARAMS,
                     scratch_types=[pltpu.SemaphoreType.DMA])(
                         *tables, *indices)


def _sc_scatter12(v0, v1, p16, dst2):
    CH = ET // NS
    NBLK = CH // W
    RT = NP // NS
    ZR = 64
    out_type = [jax.ShapeDtypeStruct((NP, 128), _f32),
                jax.ShapeDtypeStruct((NP, 128), _f32),
                jax.ShapeDtypeStruct((NP, 16), _f32)]
    scratch = [pltpu.VMEM((1, W), jnp.int32), pltpu.VMEM((W, 128), _f32),
               pltpu.VMEM((W, 16), _f32), pltpu.VMEM((ZR, 128), _f32),
               pltpu.VMEM((ZR, 16), _f32),
               pltpu.VMEM_SHARED((NP, 128), _f32),
               pltpu.VMEM_SHARED((NP, 16), _f32)]

    def body(v0_hbm, v1_hbm, p_hbm, dst_hbm, o0_hbm, o1_hbm, d_hbm, idx_v,
             val_v, pv_v, zero_v, zerop_v, acc_sh, accp_sh):
        c = lax.axis_index("c")
        s = lax.axis_index("s")

        @pl.loop(0, ZR)
        def _(r):
            zerop_v[r, :] = jnp.zeros((16,), _f32)

            @pl.loop(0, 8)
            def _(k):
                zero_v[r, pl.ds(k * 16, 16)] = jnp.zeros((16,), _f32)

        @pl.loop(0, RT // ZR)
        def _(k):
            pltpu.sync_copy(zero_v, acc_sh.at[pl.ds(s * RT + k * ZR, ZR)])

            @pl.when(c == 0)
            def _():
                pltpu.sync_copy(zerop_v,
                                accp_sh.at[pl.ds(s * RT + k * ZR, ZR)])

        plsc.subcore_barrier()
        base = s * CH

        @pl.loop(0, NBLK)
        def _(j):
            b = base + j * W
            pltpu.sync_copy(dst_hbm.at[:, pl.ds(b, W)], idx_v)

            @pl.when(c == 0)
            def _():
                pltpu.sync_copy(v0_hbm.at[pl.ds(b, W)], val_v)
                pltpu.sync_copy(p_hbm.at[pl.ds(b, W)], pv_v)
                pltpu.sync_copy(pv_v, accp_sh.at[idx_v.at[0]], add=True)

            @pl.when(c == 1)
            def _():
                pltpu.sync_copy(v1_hbm.at[pl.ds(b, W)], val_v)

            pltpu.sync_copy(val_v, acc_sh.at[idx_v.at[0]], add=True)

        plsc.subcore_barrier()

        @pl.when(c == 0)
        def _():
            pltpu.sync_copy(acc_sh.at[pl.ds(s * RT, RT)],
                            o0_hbm.at[pl.ds(s * RT, RT)])
            pltpu.sync_copy(accp_sh.at[pl.ds(s * RT, RT)],
                            d_hbm.at[pl.ds(s * RT, RT)])

        @pl.when(c == 1)
        def _():
            pltpu.sync_copy(acc_sh.at[pl.ds(s * RT, RT)],
                            o1_hbm.at[pl.ds(s * RT, RT)])

    return pl.kernel(body, out_type=out_type, mesh=_MESH,
                     compiler_params=_SC_PARAMS,
                     scratch_types=scratch)(v0, v1, p16, dst2)


def _sc_scatter3(v16, p16, dst2):
    CH = ET // (NC * NS)
    NBLK = CH // W
    RT = NP // NS
    ZR = 128
    out_type = [jax.ShapeDtypeStruct((NP, 16), _f32)] * 4
    scratch = [pltpu.VMEM((1, W), jnp.int32), pltpu.VMEM((W, 16), _f32),
               pltpu.VMEM((W, 16), _f32), pltpu.VMEM((ZR, 16), _f32),
               pltpu.VMEM_SHARED((NP, 16), _f32),
               pltpu.VMEM_SHARED((NP, 16), _f32)]

    def body(v_hbm, p_hbm, dst_hbm, oa_hbm, ob_hbm, da_hbm, db_hbm, idx_v,
             val_v, pv_v, zero_v, acc_sh, accp_sh):
        c = lax.axis_index("c")
        s = lax.axis_index("s")

        @pl.loop(0, ZR)
        def _(r):
            zero_v[r, :] = jnp.zeros((16,), _f32)

        @pl.loop(0, RT // ZR)
        def _(k):
            pltpu.sync_copy(zero_v, acc_sh.at[pl.ds(s * RT + k * ZR, ZR)])
            pltpu.sync_copy(zero_v, accp_sh.at[pl.ds(s * RT + k * ZR, ZR)])

        plsc.subcore_barrier()
        base = (c * NS + s) * CH

        @pl.loop(0, NBLK)
        def _(j):
            b = base + j * W
            pltpu.sync_copy(dst_hbm.at[:, pl.ds(b, W)], idx_v)
            pltpu.sync_copy(v_hbm.at[pl.ds(b, W)], val_v)
            pltpu.sync_copy(p_hbm.at[pl.ds(b, W)], pv_v)
            pltpu.sync_copy(val_v, acc_sh.at[idx_v.at[0]], add=True)
            pltpu.sync_copy(pv_v, accp_sh.at[idx_v.at[0]], add=True)

        plsc.subcore_barrier()

        @pl.when(c == 0)
        def _():
            pltpu.sync_copy(acc_sh.at[pl.ds(s * RT, RT)],
                            oa_hbm.at[pl.ds(s * RT, RT)])
            pltpu.sync_copy(accp_sh.at[pl.ds(s * RT, RT)],
                            da_hbm.at[pl.ds(s * RT, RT)])

        @pl.when(c == 1)
        def _():
            pltpu.sync_copy(acc_sh.at[pl.ds(s * RT, RT)],
                            ob_hbm.at[pl.ds(s * RT, RT)])
            pltpu.sync_copy(accp_sh.at[pl.ds(s * RT, RT)],
                            db_hbm.at[pl.ds(s * RT, RT)])

    return pl.kernel(body, out_type=out_type, mesh=_MESH,
                     compiler_params=_SC_PARAMS,
                     scratch_types=scratch)(v16, p16, dst2)


def _tc_call(body, grid, in_specs, out_shapes, out_specs):
    return pl.pallas_call(body, grid=grid, in_specs=in_specs,
                          out_specs=out_specs, out_shape=out_shapes)


def _full(shape):
    nd = len(shape)
    return pl.BlockSpec(shape, lambda i: (0,) * nd)


def _node_block(cw):
    return pl.BlockSpec((2000, cw), lambda i: (i, 0))


def _edge_block(cw):
    return pl.BlockSpec((2048, cw), lambda i: (i, 0))


def _prep_common(hb, wl_ref, bl_ref, wr_ref, br_ref, am_ref, xl_ref, xr_ref,
                 ms_ref, B):
    xl = jnp.dot(hb, wl_ref[...], preferred_element_type=_f32) + bl_ref[...]
    xr = jnp.dot(hb, wr_ref[...], preferred_element_type=_f32) + br_ref[...]
    ms = jnp.dot(_leaky(xl + xr), am_ref[...], preferred_element_type=_f32)
    xl_ref[...] = xl.astype(jnp.bfloat16)
    xr_ref[...] = xr.astype(jnp.bfloat16)
    ms_ref[...] = jnp.concatenate([ms, jnp.zeros((B, 12), _f32)], axis=1)


def _tc_node_prep(x, Wl, bl, Wr, br, att_mat):
    B = 2000
    D = x.shape[1]

    def body(x_ref, wl_ref, bl_ref, wr_ref, br_ref, am_ref, xl_ref, xr_ref,
             ms_ref):
        _prep_common(x_ref[...], wl_ref, bl_ref, wr_ref, br_ref, am_ref,
                     xl_ref, xr_ref, ms_ref, B)

    return _tc_call(
        body, (N // B,),
        [pl.BlockSpec((B, D), lambda i: (i, 0)), _full((D, HF)), _full((HF,)),
         _full((D, HF)), _full((HF,)), _full((HF, 4))],
        [jax.ShapeDtypeStruct((N, HF), jnp.bfloat16),
         jax.ShapeDtypeStruct((N, HF), jnp.bfloat16),
         jax.ShapeDtypeStruct((N, 16), _f32)],
        [_node_block(HF), _node_block(HF), _node_block(16)],
    )(x, Wl, bl, Wr, br, att_mat)


def _hidden_from_parts(o0_ref, o1_ref, d_ref, b_ref, B):
    d4 = d_ref[...][:, 0:4] + 1e-16
    den0 = jnp.repeat(d4[:, 0:2], 64, axis=1)
    den1 = jnp.repeat(d4[:, 2:4], 64, axis=1)
    h = jnp.concatenate([o0_ref[...] / den0, o1_ref[...] / den1], axis=1)
    return _elu(h + b_ref[...])


def _tc_node_prep2(o0, o1, d, bias, Wl, bl, Wr, br, att_mat):
    B = 2000

    def body(o0_ref, o1_ref, d_ref, b_ref, wl_ref, bl_ref, wr_ref, br_ref,
             am_ref, xl_ref, xr_ref, ms_ref):
        hb = _hidden_from_parts(o0_ref, o1_ref, d_ref, b_ref, B)
        _prep_common(hb, wl_ref, bl_ref, wr_ref, br_ref, am_ref, xl_ref,
                     xr_ref, ms_ref, B)

    return _tc_call(
        body, (N // B,),
        [_node_block(128), _node_block(128), _node_block(16), _full((HF,)),
         _full((HF, HF)), _full((HF,)), _full((HF, HF)), _full((HF,)),
         _full((HF, 4))],
        [jax.ShapeDtypeStruct((N, HF), jnp.bfloat16),
         jax.ShapeDtypeStruct((N, HF), jnp.bfloat16),
         jax.ShapeDtypeStruct((N, 16), _f32)],
        [_node_block(HF), _node_block(HF), _node_block(16)],
    )(o0, o1, d, bias, Wl, bl, Wr, br, att_mat)


def _tc_edge_pv(xls, xrd, msd, att_mat):
    B = 2048

    def body(xls_ref, xrd_ref, msd_ref, am_ref, p_ref, v0_ref, v1_ref):
        xls = xls_ref[...].astype(_f32)
        e = _leaky(xls + xrd_ref[...].astype(_f32))
        logits = jnp.dot(e, am_ref[...], preferred_element_type=_f32)
        p = jnp.exp(logits - msd_ref[...][:, 0:4])
        p_ref[...] = jnp.concatenate([p, jnp.zeros((B, 12), _f32)], axis=1)
        v = (xls.reshape(B, 4, 64) * p.reshape(B, 4, 1)).reshape(B, HF)
        v0_ref[...] = v[:, 0:128]
        v1_ref[...] = v[:, 128:256]

    return _tc_call(
        body, (ET // B,),
        [_edge_block(HF), _edge_block(HF), _edge_block(16), _full((HF, 4))],
        [jax.ShapeDtypeStruct((ET, 16), _f32),
         jax.ShapeDtypeStruct((ET, 128), _f32),
         jax.ShapeDtypeStruct((ET, 128), _f32)],
        [_edge_block(16), _edge_block(128), _edge_block(128)],
    )(xls, xrd, msd, att_mat)


def _tc_node_prep3(o0, o1, d, bias, Wl3, bl3, Wr3, br3, att3):
    B = 2000

    def body(o0_ref, o1_ref, d_ref, b_ref, wl_ref, bl_ref, wr_ref, br_ref,
             a_ref, ts_ref, td_ref):
        hb = _hidden_from_parts(o0_ref, o1_ref, d_ref, b_ref, B)
        xl = jnp.dot(hb, wl_ref[...], preferred_element_type=_f32) + bl_ref[...]
        xr = jnp.dot(hb, wr_ref[...], preferred_element_type=_f32) + br_ref[...]
        lk = _leaky(xl + xr)
        ms = lk[:, 0:1] * a_ref[0, 0] + lk[:, 1:2] * a_ref[0, 1]
        ts_ref[...] = jnp.concatenate([xl, jnp.zeros((B, 14), _f32)], axis=1)
        td_ref[...] = jnp.concatenate([xr, ms, jnp.zeros((B, 13), _f32)],
                                      axis=1)

    return _tc_call(
        body, (N // B,),
        [_node_block(128), _node_block(128), _node_block(16), _full((HF,)),
         _full((HF, 2)), _full((2,)), _full((HF, 2)), _full((2,)),
         _full((1, 2))],
        [jax.ShapeDtypeStruct((N, 16), _f32),
         jax.ShapeDtypeStruct((N, 16), _f32)],
        [_node_block(16), _node_block(16)],
    )(o0, o1, d, bias, Wl3, bl3, Wr3, br3, att3)


def _tc_edge_pv3(ts, td, att3):
    B = 2048

    def body(ts_ref, td_ref, a_ref, p_ref, v_ref):
        tsb = ts_ref[...]
        tdb = td_ref[...]
        e = _leaky(tsb[:, 0:2] + tdb[:, 0:2])
        logit = e[:, 0:1] * a_ref[0, 0] + e[:, 1:2] * a_ref[0, 1]
        p = jnp.exp(logit - tdb[:, 2:3])
        p_ref[...] = jnp.concatenate([p, jnp.zeros((B, 15), _f32)], axis=1)
        v = tsb[:, 0:2] * p
        v_ref[...] = jnp.concatenate([v, jnp.zeros((B, 14), _f32)], axis=1)

    return _tc_call(
        body, (ET // B,),
        [_edge_block(16), _edge_block(16), _full((1, 2))],
        [jax.ShapeDtypeStruct((ET, 16), _f32),
         jax.ShapeDtypeStruct((ET, 16), _f32)],
        [_edge_block(16), _edge_block(16)],
    )(ts, td, att3)


def _tc_final(oa, ob, da, db, bias3):
    B = 2000

    def body(oa_ref, ob_ref, da_ref, db_ref, b_ref, o_ref):
        num = oa_ref[...][:, 0:2] + ob_ref[...][:, 0:2]
        den = da_ref[...][:, 0:1] + db_ref[...][:, 0:1] + 1e-16
        o_ref[...] = num / den + b_ref[...]

    return _tc_call(
        body, (N // B,),
        [_node_block(16), _node_block(16), _node_block(16), _node_block(16),
         _full((2,))],
        jax.ShapeDtypeStruct((N, 2), _f32),
        pl.BlockSpec((B, 2), lambda i: (i, 0)),
    )(oa, ob, da, db, bias3)


def _att_mat(att):
    return (jnp.eye(4, dtype=_f32)[:, None, :] * att[:, :, None]).reshape(HF, 4)


def _edge_phase(xl, xr, ms16, src2, dst2, att_mat):
    xls, xrd, msd = _sc_gather_fused(
        (xl, xr, ms16), (src2, dst2, dst2), (HF, HF, 16),
        (jnp.bfloat16, jnp.bfloat16, _f32))
    p16, v0, v1 = _tc_edge_pv(xls, xrd, msd, att_mat)
    return _sc_scatter12(v0, v1, p16, dst2)


def kernel(x, edge_index, Wl1, bl1, Wr1, br1, att1, bias1, Wl2, bl2, Wr2, br2,
           att2, bias2, Wl3, bl3, Wr3, br3, att3, bias3):
    loop = jnp.arange(N, dtype=jnp.int32)
    pad = ET - (edge_index.shape[1] + N)
    pad_src = jnp.arange(pad, dtype=jnp.int32) % N
    pad_dst = N + (jnp.arange(pad, dtype=jnp.int32) % (NP - N))
    src2 = jnp.concatenate([edge_index[0], loop, pad_src]).reshape(1, ET)
    dst2 = jnp.concatenate([edge_index[1], loop, pad_dst]).reshape(1, ET)

    am1 = _att_mat(att1)
    am2 = _att_mat(att2)

    xl, xr, ms16 = _tc_node_prep(x, Wl1, bl1, Wr1, br1, am1)
    o0, o1, d = _edge_phase(xl, xr, ms16, src2, dst2, am1)

    xl, xr, ms16 = _tc_node_prep2(o0, o1, d, bias1, Wl2, bl2, Wr2, br2, am2)
    o0, o1, d = _edge_phase(xl, xr, ms16, src2, dst2, am2)

    ts, td = _tc_node_prep3(o0, o1, d, bias2, Wl3, bl3, Wr3, br3, att3)
    tss, tdd = _sc_gather_fused((ts, td), (src2, dst2), (16, 16),
                                (_f32, _f32))
    p16, v16 = _tc_edge_pv3(tss, tdd, att3)
    oa, ob, da, db = _sc_scatter3(v16, p16, dst2)
    return _tc_final(oa, ob, da, db, bias3)


# speedup vs baseline: 1.2928x; 1.2928x over previous
import jax
import jax.numpy as jnp
from jax import lax
from jax.experimental import pallas as pl
from jax.experimental.pallas import tpu as pltpu
from jax.experimental.pallas import tpu_sc as plsc

N = 10000
NP = 10240
ET = 172032
HF = 256
W = 128
NC, NS = 2, 16

_MESH = plsc.VectorSubcoreMesh(core_axis_name="c", subcore_axis_name="s")
_SC_PARAMS = pltpu.CompilerParams(use_tc_tiling_on_sc=False)
_f32 = jnp.float32


def _leaky(x):
    return jnp.where(x >= 0, x, 0.2 * x)


def _elu(x):
    return jnp.where(x > 0, x, jnp.exp(x) - 1.0)


def _sc_gather_fused(tables, indices, widths, dtypes):
    GW = 64 if len(tables) > 2 else 128
    n = len(tables)
    out_type = [jax.ShapeDtypeStruct((ET, w), dt)
                for w, dt in zip(widths, dtypes)]

    def body(*refs):
        t_hbms = refs[:n]
        i_hbms = refs[n:2 * n]
        o_hbms = refs[2 * n:3 * n]
        sem = refs[3 * n]

        def inner(*bufs):
            idx_vs = bufs[:n]
            out_vs = bufs[n:]
            copies = [pltpu.async_copy(t_hbms[k].at[idx_vs[k].at[0]],
                                       out_vs[k], sem) for k in range(n)]
            for cp in copies:
                cp.wait()

        pltpu.emit_pipeline(
            inner,
            grid=(ET // GW,),
            in_specs=[pl.BlockSpec((1, GW), lambda i: (0, i))] * n,
            out_specs=[pl.BlockSpec((GW, w), lambda i: (i, 0))
                       for w in widths],
            core_axis_name=("c", "s"),
            dimension_semantics=(pltpu.PARALLEL,),
        )(*i_hbms, *o_hbms)

    return pl.kernel(body, out_type=out_type, mesh=_MESH,
                     compiler_params=_SC_PARAMS,
                     scratch_types=[pltpu.SemaphoreType.DMA])(
                         *tables, *indices)


def _sc_scatter12(v0, v1, p16, dst2):
    CH = ET // NS
    NBLK = CH // W
    RT = NP // NS
    ZR = 64
    out_type = [jax.ShapeDtypeStruct((NP, 128), _f32),
                jax.ShapeDtypeStruct((NP, 128), _f32),
                jax.ShapeDtypeStruct((NP, 16), _f32)]
    scratch = [pltpu.VMEM((1, W), jnp.int32), pltpu.VMEM((W, 128), _f32),
               pltpu.VMEM((W, 16), _f32), pltpu.VMEM((ZR, 128), _f32),
               pltpu.VMEM((ZR, 16), _f32),
               pltpu.VMEM_SHARED((NP, 128), _f32),
               pltpu.VMEM_SHARED((NP, 16), _f32)]

    def body(v0_hbm, v1_hbm, p_hbm, dst_hbm, o0_hbm, o1_hbm, d_hbm, idx_v,
             val_v, pv_v, zero_v, zerop_v, acc_sh, accp_sh):
        c = lax.axis_index("c")
        s = lax.axis_index("s")

        @pl.loop(0, ZR)
        def _(r):
            zerop_v[r, :] = jnp.zeros((16,), _f32)

            @pl.loop(0, 8)
            def _(k):
                zero_v[r, pl.ds(k * 16, 16)] = jnp.zeros((16,), _f32)

        @pl.loop(0, RT // ZR)
        def _(k):
            pltpu.sync_copy(zero_v, acc_sh.at[pl.ds(s * RT + k * ZR, ZR)])

            @pl.when(c == 0)
            def _():
                pltpu.sync_copy(zerop_v,
                                accp_sh.at[pl.ds(s * RT + k * ZR, ZR)])

        plsc.subcore_barrier()
        base = s * CH

        @pl.loop(0, NBLK)
        def _(j):
            b = base + j * W
            pltpu.sync_copy(dst_hbm.at[:, pl.ds(b, W)], idx_v)

            @pl.when(c == 0)
            def _():
                pltpu.sync_copy(v0_hbm.at[pl.ds(b, W)], val_v)
                pltpu.sync_copy(p_hbm.at[pl.ds(b, W)], pv_v)
                pltpu.sync_copy(pv_v, accp_sh.at[idx_v.at[0]], add=True)

            @pl.when(c == 1)
            def _():
                pltpu.sync_copy(v1_hbm.at[pl.ds(b, W)], val_v)

            pltpu.sync_copy(val_v, acc_sh.at[idx_v.at[0]], add=True)

        plsc.subcore_barrier()

        @pl.when(c == 0)
        def _():
            pltpu.sync_copy(acc_sh.at[pl.ds(s * RT, RT)],
                            o0_hbm.at[pl.ds(s * RT, RT)])
            pltpu.sync_copy(accp_sh.at[pl.ds(s * RT, RT)],
                            d_hbm.at[pl.ds(s * RT, RT)])

        @pl.when(c == 1)
        def _():
            pltpu.sync_copy(acc_sh.at[pl.ds(s * RT, RT)],
                            o1_hbm.at[pl.ds(s * RT, RT)])

    return pl.kernel(body, out_type=out_type, mesh=_MESH,
                     compiler_params=_SC_PARAMS,
                     scratch_types=scratch)(v0, v1, p16, dst2)


def _sc_scatter3(v16, p16, dst2):
    CH = ET // (NC * NS)
    NBLK = CH // W
    RT = NP // NS
    ZR = 128
    out_type = [jax.ShapeDtypeStruct((NP, 16), _f32)] * 4
    scratch = [pltpu.VMEM((1, W), jnp.int32), pltpu.VMEM((W, 16), _f32),
               pltpu.VMEM((W, 16), _f32), pltpu.VMEM((ZR, 16), _f32),
               pltpu.VMEM_SHARED((NP, 16), _f32),
               pltpu.VMEM_SHARED((NP, 16), _f32)]

    def body(v_hbm, p_hbm, dst_hbm, oa_hbm, ob_hbm, da_hbm, db_hbm, idx_v,
             val_v, pv_v, zero_v, acc_sh, accp_sh):
        c = lax.axis_index("c")
        s = lax.axis_index("s")

        @pl.loop(0, ZR)
        def _(r):
            zero_v[r, :] = jnp.zeros((16,), _f32)

        @pl.loop(0, RT // ZR)
        def _(k):
            pltpu.sync_copy(zero_v, acc_sh.at[pl.ds(s * RT + k * ZR, ZR)])
            pltpu.sync_copy(zero_v, accp_sh.at[pl.ds(s * RT + k * ZR, ZR)])

        plsc.subcore_barrier()
        base = (c * NS + s) * CH

        @pl.loop(0, NBLK)
        def _(j):
            b = base + j * W
            pltpu.sync_copy(dst_hbm.at[:, pl.ds(b, W)], idx_v)
            pltpu.sync_copy(v_hbm.at[pl.ds(b, W)], val_v)
            pltpu.sync_copy(p_hbm.at[pl.ds(b, W)], pv_v)
            pltpu.sync_copy(val_v, acc_sh.at[idx_v.at[0]], add=True)
            pltpu.sync_copy(pv_v, accp_sh.at[idx_v.at[0]], add=True)

        plsc.subcore_barrier()

        @pl.when(c == 0)
        def _():
            pltpu.sync_copy(acc_sh.at[pl.ds(s * RT, RT)],
                            oa_hbm.at[pl.ds(s * RT, RT)])
            pltpu.sync_copy(accp_sh.at[pl.ds(s * RT, RT)],
                            da_hbm.at[pl.ds(s * RT, RT)])

        @pl.when(c == 1)
        def _():
            pltpu.sync_copy(acc_sh.at[pl.ds(s * RT, RT)],
                            ob_hbm.at[pl.ds(s * RT, RT)])
            pltpu.sync_copy(accp_sh.at[pl.ds(s * RT, RT)],
                            db_hbm.at[pl.ds(s * RT, RT)])

    return pl.kernel(body, out_type=out_type, mesh=_MESH,
                     compiler_params=_SC_PARAMS,
                     scratch_types=scratch)(v16, p16, dst2)


def _tc_call(body, grid, in_specs, out_shapes, out_specs):
    return pl.pallas_call(body, grid=grid, in_specs=in_specs,
                          out_specs=out_specs, out_shape=out_shapes)


def _full(shape):
    nd = len(shape)
    return pl.BlockSpec(shape, lambda i: (0,) * nd)


def _node_block(cw):
    return pl.BlockSpec((2000, cw), lambda i: (i, 0))


def _edge_block(cw):
    return pl.BlockSpec((2048, cw), lambda i: (i, 0))


def _prep_common(hb, wl_ref, bl_ref, wr_ref, br_ref, am_ref, xla_ref, xlb_ref,
                 xra_ref, xrb_ref, ms_ref, B):
    xl = jnp.dot(hb, wl_ref[...], preferred_element_type=_f32) + bl_ref[...]
    xr = jnp.dot(hb, wr_ref[...], preferred_element_type=_f32) + br_ref[...]
    ms = jnp.dot(_leaky(xl + xr), am_ref[...], preferred_element_type=_f32)
    xla_ref[...] = xl[:, 0:128]
    xlb_ref[...] = xl[:, 128:256]
    xra_ref[...] = xr[:, 0:128]
    xrb_ref[...] = xr[:, 128:256]
    ms_ref[...] = jnp.concatenate([ms, jnp.zeros((B, 12), _f32)], axis=1)


def _tc_node_prep(x, Wl, bl, Wr, br, att_mat):
    B = 2000
    D = x.shape[1]

    def body(x_ref, wl_ref, bl_ref, wr_ref, br_ref, am_ref, xla_ref, xlb_ref,
             xra_ref, xrb_ref, ms_ref):
        _prep_common(x_ref[...], wl_ref, bl_ref, wr_ref, br_ref, am_ref,
                     xla_ref, xlb_ref, xra_ref, xrb_ref, ms_ref, B)

    return _tc_call(
        body, (N // B,),
        [pl.BlockSpec((B, D), lambda i: (i, 0)), _full((D, HF)), _full((HF,)),
         _full((D, HF)), _full((HF,)), _full((HF, 4))],
        [jax.ShapeDtypeStruct((N, 128), _f32)] * 4 +
        [jax.ShapeDtypeStruct((N, 16), _f32)],
        [_node_block(128)] * 4 + [_node_block(16)],
    )(x, Wl, bl, Wr, br, att_mat)


def _hidden_from_parts(o0_ref, o1_ref, d_ref, b_ref, B):
    d4 = d_ref[...][:, 0:4] + 1e-16
    den0 = jnp.repeat(d4[:, 0:2], 64, axis=1)
    den1 = jnp.repeat(d4[:, 2:4], 64, axis=1)
    h = jnp.concatenate([o0_ref[...] / den0, o1_ref[...] / den1], axis=1)
    return _elu(h + b_ref[...])


def _tc_node_prep2(o0, o1, d, bias, Wl, bl, Wr, br, att_mat):
    B = 2000

    def body(o0_ref, o1_ref, d_ref, b_ref, wl_ref, bl_ref, wr_ref, br_ref,
             am_ref, xla_ref, xlb_ref, xra_ref, xrb_ref, ms_ref):
        hb = _hidden_from_parts(o0_ref, o1_ref, d_ref, b_ref, B)
        _prep_common(hb, wl_ref, bl_ref, wr_ref, br_ref, am_ref, xla_ref,
                     xlb_ref, xra_ref, xrb_ref, ms_ref, B)

    return _tc_call(
        body, (N // B,),
        [_node_block(128), _node_block(128), _node_block(16), _full((HF,)),
         _full((HF, HF)), _full((HF,)), _full((HF, HF)), _full((HF,)),
         _full((HF, 4))],
        [jax.ShapeDtypeStruct((N, 128), _f32)] * 4 +
        [jax.ShapeDtypeStruct((N, 16), _f32)],
        [_node_block(128)] * 4 + [_node_block(16)],
    )(o0, o1, d, bias, Wl, bl, Wr, br, att_mat)


def _tc_edge_pv(xls, xrd, msd, att_mat):
    B = 2048

    def body(xla_ref, xlb_ref, xra_ref, xrb_ref, msd_ref, am_ref, p_ref,
             v0_ref, v1_ref):
        xa = xla_ref[...]
        xb = xlb_ref[...]
        e = jnp.concatenate([_leaky(xa + xra_ref[...]),
                             _leaky(xb + xrb_ref[...])], axis=1)
        logits = jnp.dot(e, am_ref[...], preferred_element_type=_f32)
        p = jnp.exp(logits - msd_ref[...][:, 0:4])
        p_ref[...] = jnp.concatenate([p, jnp.zeros((B, 12), _f32)], axis=1)
        v0_ref[...] = (xa.reshape(B, 2, 64) *
                       p[:, 0:2].reshape(B, 2, 1)).reshape(B, 128)
        v1_ref[...] = (xb.reshape(B, 2, 64) *
                       p[:, 2:4].reshape(B, 2, 1)).reshape(B, 128)

    return _tc_call(
        body, (ET // B,),
        [_edge_block(128)] * 4 + [_edge_block(16), _full((HF, 4))],
        [jax.ShapeDtypeStruct((ET, 16), _f32),
         jax.ShapeDtypeStruct((ET, 128), _f32),
         jax.ShapeDtypeStruct((ET, 128), _f32)],
        [_edge_block(16), _edge_block(128), _edge_block(128)],
    )(xls[0], xls[1], xrd[0], xrd[1], msd, att_mat)


def _tc_node_prep3(o0, o1, d, bias, Wl3, bl3, Wr3, br3, att3):
    B = 2000

    def body(o0_ref, o1_ref, d_ref, b_ref, wl_ref, bl_ref, wr_ref, br_ref,
             a_ref, ts_ref, td_ref):
        hb = _hidden_from_parts(o0_ref, o1_ref, d_ref, b_ref, B)
        xl = jnp.dot(hb, wl_ref[...], preferred_element_type=_f32) + bl_ref[...]
        xr = jnp.dot(hb, wr_ref[...], preferred_element_type=_f32) + br_ref[...]
        lk = _leaky(xl + xr)
        ms = lk[:, 0:1] * a_ref[0, 0] + lk[:, 1:2] * a_ref[0, 1]
        ts_ref[...] = jnp.concatenate([xl, jnp.zeros((B, 14), _f32)], axis=1)
        td_ref[...] = jnp.concatenate([xr, ms, jnp.zeros((B, 13), _f32)],
                                      axis=1)

    return _tc_call(
        body, (N // B,),
        [_node_block(128), _node_block(128), _node_block(16), _full((HF,)),
         _full((HF, 2)), _full((2,)), _full((HF, 2)), _full((2,)),
         _full((1, 2))],
        [jax.ShapeDtypeStruct((N, 16), _f32),
         jax.ShapeDtypeStruct((N, 16), _f32)],
        [_node_block(16), _node_block(16)],
    )(o0, o1, d, bias, Wl3, bl3, Wr3, br3, att3)


def _tc_edge_pv3(ts, td, att3):
    B = 2048

    def body(ts_ref, td_ref, a_ref, p_ref, v_ref):
        tsb = ts_ref[...]
        tdb = td_ref[...]
        e = _leaky(tsb[:, 0:2] + tdb[:, 0:2])
        logit = e[:, 0:1] * a_ref[0, 0] + e[:, 1:2] * a_ref[0, 1]
        p = jnp.exp(logit - tdb[:, 2:3])
        p_ref[...] = jnp.concatenate([p, jnp.zeros((B, 15), _f32)], axis=1)
        v = tsb[:, 0:2] * p
        v_ref[...] = jnp.concatenate([v, jnp.zeros((B, 14), _f32)], axis=1)

    return _tc_call(
        body, (ET // B,),
        [_edge_block(16), _edge_block(16), _full((1, 2))],
        [jax.ShapeDtypeStruct((ET, 16), _f32),
         jax.ShapeDtypeStruct((ET, 16), _f32)],
        [_edge_block(16), _edge_block(16)],
    )(ts, td, att3)


def _tc_final(oa, ob, da, db, bias3):
    B = 2000

    def body(oa_ref, ob_ref, da_ref, db_ref, b_ref, o_ref):
        num = oa_ref[...][:, 0:2] + ob_ref[...][:, 0:2]
        den = da_ref[...][:, 0:1] + db_ref[...][:, 0:1] + 1e-16
        o_ref[...] = num / den + b_ref[...]

    return _tc_call(
        body, (N // B,),
        [_node_block(16), _node_block(16), _node_block(16), _node_block(16),
         _full((2,))],
        jax.ShapeDtypeStruct((N, 2), _f32),
        pl.BlockSpec((B, 2), lambda i: (i, 0)),
    )(oa, ob, da, db, bias3)


def _att_mat(att):
    return (jnp.eye(4, dtype=_f32)[:, None, :] * att[:, :, None]).reshape(HF, 4)


def _edge_phase(xl, xr, ms16, src2, dst2, att_mat):
    xla_, xlb_, xra_, xrb_, msd = _sc_gather_fused(
        (xl[0], xl[1], xr[0], xr[1], ms16),
        (src2, src2, dst2, dst2, dst2), (128, 128, 128, 128, 16),
        (_f32, _f32, _f32, _f32, _f32))
    p16, v0, v1 = _tc_edge_pv((xla_, xlb_), (xra_, xrb_), msd, att_mat)
    return _sc_scatter12(v0, v1, p16, dst2)


def kernel(x, edge_index, Wl1, bl1, Wr1, br1, att1, bias1, Wl2, bl2, Wr2, br2,
           att2, bias2, Wl3, bl3, Wr3, br3, att3, bias3):
    loop = jnp.arange(N, dtype=jnp.int32)
    pad = ET - (edge_index.shape[1] + N)
    pad_src = jnp.arange(pad, dtype=jnp.int32) % N
    pad_dst = N + (jnp.arange(pad, dtype=jnp.int32) % (NP - N))
    src2 = jnp.concatenate([edge_index[0], loop, pad_src]).reshape(1, ET)
    dst2 = jnp.concatenate([edge_index[1], loop, pad_dst]).reshape(1, ET)

    am1 = _att_mat(att1)
    am2 = _att_mat(att2)

    xla_, xlb_, xra_, xrb_, ms16 = _tc_node_prep(x, Wl1, bl1, Wr1, br1, am1)
    o0, o1, d = _edge_phase((xla_, xlb_), (xra_, xrb_), ms16, src2, dst2, am1)

    xla_, xlb_, xra_, xrb_, ms16 = _tc_node_prep2(o0, o1, d, bias1, Wl2, bl2,
                                                  Wr2, br2, am2)
    o0, o1, d = _edge_phase((xla_, xlb_), (xra_, xrb_), ms16, src2, dst2, am2)

    ts, td = _tc_node_prep3(o0, o1, d, bias2, Wl3, bl3, Wr3, br3, att3)
    tss, tdd = _sc_gather_fused((ts, td), (src2, dst2), (16, 16),
                                (_f32, _f32))
    p16, v16 = _tc_edge_pv3(tss, tdd, att3)
    oa, ob, da, db = _sc_scatter3(v16, p16, dst2)
    return _tc_final(oa, ob, da, db, bias3)
